# baseline (speedup 1.0000x reference)
import math

import numpy as np
import jax
import jax.numpy as jnp
from jax.experimental import pallas as pl
from jax.experimental.pallas import tpu as pltpu

_VMEM_LIMIT = 48 * 1024 * 1024


def _band_onehot(w_in, c_in, n_q, c_major):
    kdim = c_in * 25
    lanes = w_in * c_in
    p = np.zeros((5, lanes, 2, n_q, kdim), np.float32)
    for kh in range(5):
        for lane in range(lanes):
            if c_major:
                c, w = divmod(lane, w_in)
            else:
                w, c = divmod(lane, c_in)
            for dj in (0, 1):
                for q in range(n_q):
                    kw = w - 2 * q - dj
                    if 0 <= kw < 5:
                        p[kh, lane, dj, q, c * 25 + kh * 5 + kw] = 1.0
    return p.reshape(5 * lanes * 2 * n_q, kdim)


_P1 = _band_onehot(32, 3, 14, c_major=True)
_P2 = _band_onehot(14, 6, 5, c_major=False)


def _pick_tile(b, target):
    if b <= target:
        return b
    t = (target // 8) * 8
    while t >= 8:
        if b % t == 0:
            return t
        t -= 8
    return b


def _lenet_kernel(x_ref, w1_ref, w2_ref, b1_ref, b2_ref,
                  f1_ref, fb1_ref, f2_ref, fb2_ref, f3_ref, fb3_ref, o_ref,
                  xs_ref):
    tb = o_ref.shape[0]
    f32 = jnp.float32

    xv = x_ref[...]
    for m4 in range(4):
        for j in range(8):
            h = 4 * j + m4
            xs_ref[m4, j] = jnp.concatenate(
                [xv[:, c * 1024 + h * 32: c * 1024 + h * 32 + 32]
                 for c in range(3)], axis=1)

    h1 = []
    for par in (0, 1):
        acc = []
        for di in (0, 1):
            a = None
            for kh in range(5):
                r = 2 * par + di + kh
                slab = xs_ref[r % 4, r // 4:r // 4 + 7].reshape(7 * tb, 96)
                t = jnp.dot(slab, w1_ref[kh], preferred_element_type=f32)
                a = t if a is None else a + t
            acc.append(a)
        m = jnp.maximum(acc[0], acc[1])
        m = jnp.maximum(m[:, :128], m[:, 128:])
        m = jnp.maximum(m + b1_ref[0], 0.0)
        h1.append(m.astype(jnp.bfloat16).reshape(7, tb, 128))

    acc = []
    for di in (0, 1):
        a = None
        for kh in range(5):
            r = di + kh
            slab = h1[r % 2][r // 2:r // 2 + 5].reshape(5 * tb, 128)
            t = jnp.dot(slab, w2_ref[kh], preferred_element_type=f32)
            a = t if a is None else a + t
        acc.append(a)
    m = jnp.maximum(acc[0], acc[1])
    m = jnp.maximum(m[:, :128], m[:, 128:])
    h2 = jnp.maximum(m + b2_ref[0], 0.0).reshape(5, tb, 128)

    a = None
    for p in range(5):
        t = jnp.dot(h2[p], f1_ref[p], preferred_element_type=f32)
        a = t if a is None else a + t
    h = jnp.maximum(a + fb1_ref[0], 0.0)
    h = jnp.maximum(jnp.dot(h, f2_ref[...], preferred_element_type=f32)
                    + fb2_ref[0], 0.0)
    o_ref[...] = jnp.dot(h, f3_ref[...], preferred_element_type=f32) + fb3_ref[0]


def kernel(x, c1_w, c1_b, c2_w, c2_b, f1_w, f1_b, f2_w, f2_b, f3_w, f3_b):
    B = x.shape[0]
    bf16 = jnp.bfloat16

    x2d = x.astype(bf16).reshape(B, 3 * 32 * 32)

    w1 = (jnp.dot(jnp.asarray(_P1), c1_w)
             .reshape(5, 96, 2, 84))
    w1 = jnp.pad(w1, ((0, 0), (0, 0), (0, 0), (0, 44)))
    w1 = w1.reshape(5, 96, 256).astype(bf16)
    w2 = (jnp.dot(jnp.asarray(_P2), c2_w)
             .reshape(5, 84, 2, 80))
    w2 = jnp.pad(w2, ((0, 0), (0, 0), (0, 0), (0, 48)))
    w2 = w2.reshape(5, 84, 256)
    w2 = jnp.pad(w2, ((0, 0), (0, 44), (0, 0))).astype(bf16)

    b1 = jnp.pad(jnp.tile(c1_b, (1, 14)), ((0, 0), (0, 128 - 84)))
    b2 = jnp.pad(jnp.tile(c2_b, (1, 5)), ((0, 0), (0, 128 - 80)))
    f1p = jnp.pad(f1_w.reshape(5, 80, 120), ((0, 0), (0, 48), (0, 0)))

    tb = _pick_tile(B, 512)
    grid = (B // tb,)
    full = lambda shape: pl.BlockSpec(shape, lambda i: (0,) * len(shape))
    out = pl.pallas_call(
        _lenet_kernel,
        out_shape=jax.ShapeDtypeStruct((B, 10), jnp.float32),
        grid=grid,
        in_specs=[pl.BlockSpec((tb, 3072), lambda i: (i, 0)),
                  full((5, 96, 256)),
                  full((5, 128, 256)),
                  full((1, 128)),
                  full((1, 128)),
                  full((5, 128, 120)),
                  full((1, 120)),
                  full((120, 84)),
                  full((1, 84)),
                  full((84, 10)),
                  full((1, 10))],
        out_specs=pl.BlockSpec((tb, 10), lambda i: (i, 0)),
        scratch_shapes=[pltpu.VMEM((4, 8, tb, 96), bf16)],
        compiler_params=pltpu.CompilerParams(
            dimension_semantics=("arbitrary",),
            vmem_limit_bytes=_VMEM_LIMIT),
    )(x2d, w1, w2, b1, b2, f1p, f1_b, f2_w, f2_b, f3_w, f3_b)
    return out


# speedup vs baseline: 1.0131x; 1.0131x over previous
import math

import numpy as np
import jax
import jax.numpy as jnp
from jax.experimental import pallas as pl
from jax.experimental.pallas import tpu as pltpu

_VMEM_LIMIT = 48 * 1024 * 1024


def _band_onehot(w_in, c_in, n_q, c_major):
    kdim = c_in * 25
    lanes = w_in * c_in
    p = np.zeros((5, lanes, 2, n_q, kdim), np.float32)
    for kh in range(5):
        for lane in range(lanes):
            if c_major:
                c, w = divmod(lane, w_in)
            else:
                w, c = divmod(lane, c_in)
            for dj in (0, 1):
                for q in range(n_q):
                    kw = w - 2 * q - dj
                    if 0 <= kw < 5:
                        p[kh, lane, dj, q, c * 25 + kh * 5 + kw] = 1.0
    return p.reshape(5 * lanes * 2 * n_q, kdim)


_P1 = _band_onehot(32, 3, 14, c_major=True)
_P2 = _band_onehot(14, 6, 5, c_major=False)


def _pick_tile(b, target):
    if b <= target:
        return b
    t = (target // 8) * 8
    while t >= 8:
        if b % t == 0:
            return t
        t -= 8
    return b


def _lenet_kernel(x_ref, w1_ref, w2_ref, b1_ref, b2_ref,
                  f1_ref, fb1_ref, f2_ref, fb2_ref, f3_ref, fb3_ref, o_ref,
                  xs_ref):
    tb = o_ref.shape[0]
    f32 = jnp.float32

    xv = x_ref[...]
    for m4 in range(4):
        for j in range(8):
            h = 4 * j + m4
            xs_ref[m4, j] = jnp.concatenate(
                [xv[:, c * 1024 + h * 32: c * 1024 + h * 32 + 32]
                 for c in range(3)], axis=1).astype(jnp.bfloat16)

    h1 = []
    for par in (0, 1):
        acc = []
        for di in (0, 1):
            a = None
            for kh in range(5):
                r = 2 * par + di + kh
                slab = xs_ref[r % 4, r // 4:r // 4 + 7].reshape(7 * tb, 96)
                t = jnp.dot(slab, w1_ref[kh], preferred_element_type=f32)
                a = t if a is None else a + t
            acc.append(a)
        m = jnp.maximum(acc[0], acc[1])
        m = jnp.maximum(m[:, :128], m[:, 128:])
        m = jnp.maximum(m + b1_ref[0], 0.0)
        h1.append(m.astype(jnp.bfloat16).reshape(7, tb, 128))

    acc = []
    for di in (0, 1):
        a = None
        for kh in range(5):
            r = di + kh
            slab = h1[r % 2][r // 2:r // 2 + 5].reshape(5 * tb, 128)
            t = jnp.dot(slab, w2_ref[kh], preferred_element_type=f32)
            a = t if a is None else a + t
        acc.append(a)
    m = jnp.maximum(acc[0], acc[1])
    m = jnp.maximum(m[:, :128], m[:, 128:])
    h2 = jnp.maximum(m + b2_ref[0], 0.0).reshape(5, tb, 128)

    a = None
    for p in range(5):
        t = jnp.dot(h2[p], f1_ref[p], preferred_element_type=f32)
        a = t if a is None else a + t
    h = jnp.maximum(a + fb1_ref[0], 0.0)
    h = jnp.maximum(jnp.dot(h, f2_ref[...], preferred_element_type=f32)
                    + fb2_ref[0], 0.0)
    o_ref[...] = jnp.dot(h, f3_ref[...], preferred_element_type=f32) + fb3_ref[0]


def kernel(x, c1_w, c1_b, c2_w, c2_b, f1_w, f1_b, f2_w, f2_b, f3_w, f3_b):
    B = x.shape[0]
    bf16 = jnp.bfloat16

    x2d = x.reshape(B, 3 * 32 * 32)

    w1 = (jnp.dot(jnp.asarray(_P1), c1_w)
             .reshape(5, 96, 2, 84))
    w1 = jnp.pad(w1, ((0, 0), (0, 0), (0, 0), (0, 44)))
    w1 = w1.reshape(5, 96, 256).astype(bf16)
    w2 = (jnp.dot(jnp.asarray(_P2), c2_w)
             .reshape(5, 84, 2, 80))
    w2 = jnp.pad(w2, ((0, 0), (0, 0), (0, 0), (0, 48)))
    w2 = w2.reshape(5, 84, 256)
    w2 = jnp.pad(w2, ((0, 0), (0, 44), (0, 0))).astype(bf16)

    b1 = jnp.pad(jnp.tile(c1_b, (1, 14)), ((0, 0), (0, 128 - 84)))
    b2 = jnp.pad(jnp.tile(c2_b, (1, 5)), ((0, 0), (0, 128 - 80)))
    f1p = jnp.pad(f1_w.reshape(5, 80, 120), ((0, 0), (0, 48), (0, 0)))

    tb = _pick_tile(B, 256)
    grid = (B // tb,)
    full = lambda shape: pl.BlockSpec(shape, lambda i: (0,) * len(shape))
    out = pl.pallas_call(
        _lenet_kernel,
        out_shape=jax.ShapeDtypeStruct((B, 10), jnp.float32),
        grid=grid,
        in_specs=[pl.BlockSpec((tb, 3072), lambda i: (i, 0)),
                  full((5, 96, 256)),
                  full((5, 128, 256)),
                  full((1, 128)),
                  full((1, 128)),
                  full((5, 128, 120)),
                  full((1, 120)),
                  full((120, 84)),
                  full((1, 84)),
                  full((84, 10)),
                  full((1, 10))],
        out_specs=pl.BlockSpec((tb, 10), lambda i: (i, 0)),
        scratch_shapes=[pltpu.VMEM((4, 8, tb, 96), bf16)],
        compiler_params=pltpu.CompilerParams(
            dimension_semantics=("arbitrary",),
            vmem_limit_bytes=_VMEM_LIMIT),
    )(x2d, w1, w2, b1, b2, f1p, f1_b, f2_w, f2_b, f3_w, f3_b)
    return out


# speedup vs baseline: 1.0250x; 1.0117x over previous
import numpy as np
import jax
import jax.numpy as jnp
from jax.experimental import pallas as pl
from jax.experimental.pallas import tpu as pltpu

_VMEM_LIMIT = 48 * 1024 * 1024


def _band_onehot(w_in, c_in, n_q, c_major):
    kdim = c_in * 25
    lanes = w_in * c_in
    p = np.zeros((5, lanes, 2, n_q, kdim), np.float32)
    for kh in range(5):
        for lane in range(lanes):
            if c_major:
                c, w = divmod(lane, w_in)
            else:
                w, c = divmod(lane, c_in)
            for dj in (0, 1):
                for q in range(n_q):
                    kw = w - 2 * q - dj
                    if 0 <= kw < 5:
                        p[kh, lane, dj, q, c * 25 + kh * 5 + kw] = 1.0
    return p.reshape(5 * lanes * 2 * n_q, kdim)


_P1 = _band_onehot(32, 3, 14, c_major=True)
_P2 = _band_onehot(14, 6, 5, c_major=False)


def _pick_tile(b, target):
    if b <= target:
        return b
    t = (target // 8) * 8
    while t >= 8:
        if b % t == 0:
            return t
        t -= 8
    return b


def _lenet_kernel(x_ref, w1_ref, w2_ref, b1_ref, b2_ref,
                  f1_ref, fb1_ref, f2_ref, fb2_ref, f3_ref, fb3_ref, o_ref,
                  xs_ref):
    tb = o_ref.shape[0]
    f32 = jnp.float32

    xv = x_ref[...]
    for m4 in range(4):
        for j in range(8):
            h = 4 * j + m4
            xs_ref[m4, j] = jnp.concatenate(
                [xv[:, c * 1024 + h * 32: c * 1024 + h * 32 + 32]
                 for c in range(3)], axis=1).astype(jnp.bfloat16)

    h1 = []
    for par in (0, 1):
        acc = []
        for di in (0, 1):
            a = None
            for kh in range(5):
                r = 2 * par + di + kh
                slab = xs_ref[r % 4, r // 4:r // 4 + 7].reshape(7 * tb, 96)
                t = jnp.dot(slab, w1_ref[kh], preferred_element_type=f32)
                a = t if a is None else a + t
            acc.append(a)
        m = jnp.maximum(acc[0], acc[1])
        m = jnp.maximum(m[:, :128], m[:, 128:])
        m = jnp.maximum(m + b1_ref[0], 0.0)
        h1.append(m.astype(jnp.bfloat16).reshape(7, tb, 128))

    acc = []
    for di in (0, 1):
        a = None
        for kh in range(5):
            r = di + kh
            slab = h1[r % 2][r // 2:r // 2 + 5].reshape(5 * tb, 128)
            t = jnp.dot(slab, w2_ref[kh], preferred_element_type=f32)
            a = t if a is None else a + t
        acc.append(a)
    m = jnp.maximum(acc[0], acc[1])
    m = jnp.maximum(m[:, :128], m[:, 128:])
    h2 = jnp.maximum(m + b2_ref[0], 0.0).reshape(5, tb, 128)

    a = None
    for p in range(5):
        t = jnp.dot(h2[p], f1_ref[p], preferred_element_type=f32)
        a = t if a is None else a + t
    h = jnp.maximum(a + fb1_ref[0], 0.0)
    h = jnp.maximum(jnp.dot(h, f2_ref[...], preferred_element_type=f32)
                    + fb2_ref[0], 0.0)
    o_ref[...] = jnp.dot(h, f3_ref[...], preferred_element_type=f32) + fb3_ref[0]


def kernel(x, c1_w, c1_b, c2_w, c2_b, f1_w, f1_b, f2_w, f2_b, f3_w, f3_b):
    B = x.shape[0]
    bf16 = jnp.bfloat16

    x2d = x.reshape(B, 3 * 32 * 32)

    w1 = (jnp.dot(jnp.asarray(_P1), c1_w)
             .reshape(5, 96, 2, 84))
    w1 = jnp.pad(w1, ((0, 0), (0, 0), (0, 0), (0, 44)))
    w1 = w1.reshape(5, 96, 256).astype(bf16)
    w2 = (jnp.dot(jnp.asarray(_P2), c2_w)
             .reshape(5, 84, 2, 80))
    w2 = jnp.pad(w2, ((0, 0), (0, 0), (0, 0), (0, 48)))
    w2 = w2.reshape(5, 84, 256)
    w2 = jnp.pad(w2, ((0, 0), (0, 44), (0, 0))).astype(bf16)

    b1 = jnp.pad(jnp.tile(c1_b, (1, 14)), ((0, 0), (0, 128 - 84)))
    b2 = jnp.pad(jnp.tile(c2_b, (1, 5)), ((0, 0), (0, 128 - 80)))
    f1p = jnp.pad(f1_w.reshape(5, 80, 120), ((0, 0), (0, 48), (0, 0)))

    tb = _pick_tile(B, 512)
    grid = (B // tb,)
    full = lambda shape: pl.BlockSpec(shape, lambda i: (0,) * len(shape))
    out = pl.pallas_call(
        _lenet_kernel,
        out_shape=jax.ShapeDtypeStruct((B, 10), jnp.float32),
        grid=grid,
        in_specs=[pl.BlockSpec((tb, 3072), lambda i: (i, 0)),
                  full((5, 96, 256)),
                  full((5, 128, 256)),
                  full((1, 128)),
                  full((1, 128)),
                  full((5, 128, 120)),
                  full((1, 120)),
                  full((120, 84)),
                  full((1, 84)),
                  full((84, 10)),
                  full((1, 10))],
        out_specs=pl.BlockSpec((tb, 10), lambda i: (i, 0)),
        scratch_shapes=[pltpu.VMEM((4, 8, tb, 96), bf16)],
        compiler_params=pltpu.CompilerParams(
            dimension_semantics=("arbitrary",),
            vmem_limit_bytes=_VMEM_LIMIT),
    )(x2d, w1, w2, b1, b2, f1p, f1_b, f2_w, f2_b, f3_w, f3_b)
    return out
